# baseline (device time: 23754 ns/iter reference)
import jax
import jax.numpy as jnp
from jax import lax
from jax.experimental import pallas as pl
from jax.experimental.pallas import tpu as pltpu

CHUNKS = (32, 64, 128, 160, 160, 160, 160, 160)
C = len(CHUNKS)
OFFS = tuple(sum(CHUNKS[:i]) for i in range(C))


def kernel(x):
    m, n = x.shape
    half = m // 2

    def body(x_ref, out_ref, xv, send1, recv1, hbuf, ldma, odma, s1, r1, s2, r2):
        my_x = lax.axis_index("x")
        my_y = lax.axis_index("y")
        xp = (1 - my_x, my_y)
        yp = (my_x, 1 - my_y)

        dmas = []
        for c in range(C):
            d = pltpu.make_async_copy(
                x_ref.at[pl.ds(my_y * half + OFFS[c], CHUNKS[c])],
                xv.at[pl.ds(OFFS[c], CHUNKS[c])],
                ldma.at[c],
            )
            d.start()
            dmas.append(d)

        barrier_sem = pltpu.get_barrier_semaphore()
        for nbr in (xp, yp):
            pl.semaphore_signal(
                barrier_sem, inc=1, device_id=nbr,
                device_id_type=pl.DeviceIdType.MESH,
            )
        pl.semaphore_wait(barrier_sem, 2)

        rdma1 = []
        for c in range(C):
            sl = pl.ds(OFFS[c], CHUNKS[c])
            dmas[c].wait()
            send1[sl, :] = xv[sl, :].astype(jnp.bfloat16)
            d = pltpu.make_async_remote_copy(
                src_ref=send1.at[sl],
                dst_ref=recv1.at[sl],
                send_sem=s1.at[c],
                recv_sem=r1.at[c],
                device_id=xp,
                device_id_type=pl.DeviceIdType.MESH,
            )
            d.start()
            rdma1.append(d)

        rdma2 = []
        odmas = []
        for c in range(C):
            sl = pl.ds(OFFS[c], CHUNKS[c])
            rows = pl.ds(my_y * half + OFFS[c], CHUNKS[c])
            rdma1[c].wait_recv()
            hbuf[sl, :] = send1[sl, :] + recv1[sl, :]
            d = pltpu.make_async_remote_copy(
                src_ref=hbuf.at[sl],
                dst_ref=out_ref.at[rows],
                send_sem=s2.at[c],
                recv_sem=r2.at[c],
                device_id=yp,
                device_id_type=pl.DeviceIdType.MESH,
            )
            d.start()
            rdma2.append(d)
            od = pltpu.make_async_copy(hbuf.at[sl], out_ref.at[rows], odma.at[c])
            od.start()
            odmas.append(od)

        for c in range(C):
            other = pl.ds((1 - my_y) * half + OFFS[c], CHUNKS[c])
            recv = pltpu.make_async_remote_copy(
                src_ref=hbuf.at[pl.ds(OFFS[c], CHUNKS[c])],
                dst_ref=out_ref.at[other],
                send_sem=s2.at[c],
                recv_sem=r2.at[c],
                device_id=yp,
                device_id_type=pl.DeviceIdType.MESH,
            )
            recv.wait_recv()

        for c in range(C):
            odmas[c].wait()
            rdma1[c].wait_send()
            rdma2[c].wait_send()

    return pl.pallas_call(
        body,
        out_shape=jax.ShapeDtypeStruct((m, n), jnp.bfloat16),
        in_specs=[pl.BlockSpec(memory_space=pl.ANY)],
        out_specs=pl.BlockSpec(memory_space=pl.ANY),
        scratch_shapes=[
            pltpu.VMEM((half, n), jnp.float32),
            pltpu.VMEM((half, n), jnp.bfloat16),
            pltpu.VMEM((half, n), jnp.bfloat16),
            pltpu.VMEM((half, n), jnp.bfloat16),
            pltpu.SemaphoreType.DMA((C,)),
            pltpu.SemaphoreType.DMA((C,)),
            pltpu.SemaphoreType.DMA((C,)),
            pltpu.SemaphoreType.DMA((C,)),
            pltpu.SemaphoreType.DMA((C,)),
            pltpu.SemaphoreType.DMA((C,)),
        ],
        compiler_params=pltpu.CompilerParams(collective_id=0),
    )(x)


# device time: 21426 ns/iter; 1.1087x vs baseline; 1.1087x over previous
import jax
import jax.numpy as jnp
from jax import lax
from jax.experimental import pallas as pl
from jax.experimental.pallas import tpu as pltpu

CHUNKS = (32, 64, 128, 160, 160, 160, 160, 160)
C = len(CHUNKS)
OFFS = tuple(sum(CHUNKS[:i]) for i in range(C))


def kernel(x):
    m, n = x.shape
    half = m // 2
    x = pltpu.with_memory_space_constraint(x, pltpu.MemorySpace.HBM)

    def body(x_ref, out_ref, xv, send1, recv1, hbuf, ldma, odma, s1, r1, s2, r2):
        my_x = lax.axis_index("x")
        my_y = lax.axis_index("y")
        xp = (1 - my_x, my_y)
        yp = (my_x, 1 - my_y)

        dmas = []
        for c in range(C):
            d = pltpu.make_async_copy(
                x_ref.at[pl.ds(my_y * half + OFFS[c], CHUNKS[c])],
                xv.at[pl.ds(OFFS[c], CHUNKS[c])],
                ldma.at[c],
            )
            d.start()
            dmas.append(d)

        barrier_sem = pltpu.get_barrier_semaphore()
        for nbr in (xp, yp):
            pl.semaphore_signal(
                barrier_sem, inc=1, device_id=nbr,
                device_id_type=pl.DeviceIdType.MESH,
            )
        pl.semaphore_wait(barrier_sem, 2)

        rdma1 = []
        for c in range(C):
            sl = pl.ds(OFFS[c], CHUNKS[c])
            dmas[c].wait()
            send1[sl, :] = xv[sl, :].astype(jnp.bfloat16)
            d = pltpu.make_async_remote_copy(
                src_ref=send1.at[sl],
                dst_ref=recv1.at[sl],
                send_sem=s1.at[c],
                recv_sem=r1.at[c],
                device_id=xp,
                device_id_type=pl.DeviceIdType.MESH,
            )
            d.start()
            rdma1.append(d)

        rdma2 = []
        odmas = []
        for c in range(C):
            sl = pl.ds(OFFS[c], CHUNKS[c])
            rows = pl.ds(my_y * half + OFFS[c], CHUNKS[c])
            rdma1[c].wait_recv()
            hbuf[sl, :] = send1[sl, :] + recv1[sl, :]
            d = pltpu.make_async_remote_copy(
                src_ref=hbuf.at[sl],
                dst_ref=out_ref.at[rows],
                send_sem=s2.at[c],
                recv_sem=r2.at[c],
                device_id=yp,
                device_id_type=pl.DeviceIdType.MESH,
            )
            d.start()
            rdma2.append(d)
            od = pltpu.make_async_copy(hbuf.at[sl], out_ref.at[rows], odma.at[c])
            od.start()
            odmas.append(od)

        for c in range(C):
            other = pl.ds((1 - my_y) * half + OFFS[c], CHUNKS[c])
            recv = pltpu.make_async_remote_copy(
                src_ref=hbuf.at[pl.ds(OFFS[c], CHUNKS[c])],
                dst_ref=out_ref.at[other],
                send_sem=s2.at[c],
                recv_sem=r2.at[c],
                device_id=yp,
                device_id_type=pl.DeviceIdType.MESH,
            )
            recv.wait_recv()

        for c in range(C):
            odmas[c].wait()
            rdma1[c].wait_send()
            rdma2[c].wait_send()

    return pl.pallas_call(
        body,
        out_shape=jax.ShapeDtypeStruct((m, n), jnp.bfloat16),
        in_specs=[pl.BlockSpec(memory_space=pl.ANY)],
        out_specs=pl.BlockSpec(memory_space=pl.ANY),
        scratch_shapes=[
            pltpu.VMEM((half, n), jnp.float32),
            pltpu.VMEM((half, n), jnp.bfloat16),
            pltpu.VMEM((half, n), jnp.bfloat16),
            pltpu.VMEM((half, n), jnp.bfloat16),
            pltpu.SemaphoreType.DMA((C,)),
            pltpu.SemaphoreType.DMA((C,)),
            pltpu.SemaphoreType.DMA((C,)),
            pltpu.SemaphoreType.DMA((C,)),
            pltpu.SemaphoreType.DMA((C,)),
            pltpu.SemaphoreType.DMA((C,)),
        ],
        compiler_params=pltpu.CompilerParams(collective_id=0),
    )(x)
